# baseline (speedup 1.0000x reference)
import functools

import jax
import jax.numpy as jnp
from jax.experimental import pallas as pl
from jax.experimental.pallas import tpu as pltpu

_USER = 4096
_N = 8192
_D = 16
_BM = 256
_NB = _N // _BM
_NBU = _USER // _BM


def _proj_body(uf_ref, u1w_ref, usf_ref, u2w_ref, itf_ref, iw_ref, out_ref):
    ue1 = jnp.dot(uf_ref[...], u1w_ref[...], preferred_element_type=jnp.float32)
    ue2 = jnp.dot(usf_ref[...], u2w_ref[...], preferred_element_type=jnp.float32)
    ie = jnp.dot(itf_ref[...], iw_ref[...], preferred_element_type=jnp.float32)
    ue = jnp.concatenate([ue1, ue2], axis=1)
    out_ref[...] = jnp.concatenate([ue, ie], axis=0).astype(jnp.bfloat16)


def _layer_body(a_ref, x_ref, w_ref, alpha_ref, emb_ref, ego_ref, abf_ref,
                *, cast_a, emit_ego):
    a = a_ref[...]
    if cast_a:
        a = a.astype(jnp.bfloat16)
        abf_ref[...] = a
    acc = jnp.dot(a, x_ref[...], preferred_element_type=jnp.float32)
    alpha = alpha_ref[0, 0]
    emb = jnp.where(acc >= 0, acc, alpha * acc)
    emb_ref[...] = emb
    if emit_ego:
        ego = jnp.dot(emb, w_ref[0], preferred_element_type=jnp.float32)
        ego_ref[...] = ego.astype(jnp.bfloat16)


def _row_spec(i):
    return (i, 0)


def _const_spec(i):
    return (0, 0)


def _w_spec(i):
    return (jnp.where(i < _NBU, 0, 1), 0, 0)


def _layer_call(a, x, w_stack, alpha, *, cast_a, emit_ego):
    in_specs = [
        pl.BlockSpec((_BM, _N), _row_spec),
        pl.BlockSpec((_N, _D), _const_spec),
    ]
    operands = [a, x]
    if emit_ego:
        in_specs.append(pl.BlockSpec((1, _D, _D), _w_spec))
        operands.append(w_stack)
    else:
        in_specs.append(pl.BlockSpec((1, _D, _D), lambda i: (0, 0, 0)))
        operands.append(jnp.zeros((1, _D, _D), jnp.float32))
    in_specs.append(pl.BlockSpec((1, 1), _const_spec))
    operands.append(alpha)

    out_shape = [jax.ShapeDtypeStruct((_N, _D), jnp.float32)]
    out_specs = [pl.BlockSpec((_BM, _D), _row_spec)]
    if emit_ego:
        out_shape.append(jax.ShapeDtypeStruct((_N, _D), jnp.bfloat16))
        out_specs.append(pl.BlockSpec((_BM, _D), _row_spec))
    if cast_a:
        out_shape.append(jax.ShapeDtypeStruct((_N, _N), jnp.bfloat16))
        out_specs.append(pl.BlockSpec((_BM, _N), _row_spec))

    def body(a_ref, x_ref, w_ref, alpha_ref, *outs):
        emb_ref = outs[0]
        ego_ref = outs[1] if emit_ego else None
        abf_ref = outs[-1] if cast_a else None
        _layer_body(a_ref, x_ref, w_ref, alpha_ref, emb_ref, ego_ref, abf_ref,
                    cast_a=cast_a, emit_ego=emit_ego)

    return pl.pallas_call(
        body,
        grid=(_NB,),
        in_specs=in_specs,
        out_specs=out_specs,
        out_shape=out_shape,
        compiler_params=pltpu.CompilerParams(
            dimension_semantics=("parallel",)),
    )(*operands)


def kernel(user_social_feat, user_feat, item_feat, raitng_adj,
           user1_w, user2_w, item_w, user_w1, item_w1, user_w2, item_w2,
           prelu_a):
    ego0 = pl.pallas_call(
        _proj_body,
        out_shape=jax.ShapeDtypeStruct((_N, _D), jnp.bfloat16),
    )(user_feat, user1_w, user_social_feat, user2_w, item_feat, item_w)

    alpha = jnp.reshape(prelu_a, (1, 1))
    w1 = jnp.stack([user_w1, item_w1])
    w2 = jnp.stack([user_w2, item_w2])

    emb0, ego1, a_bf = _layer_call(raitng_adj, ego0, w1, alpha,
                                   cast_a=True, emit_ego=True)
    emb1, ego2 = _layer_call(a_bf, ego1, w2, alpha,
                             cast_a=False, emit_ego=True)
    (emb2,) = _layer_call(a_bf, ego2, None, alpha,
                          cast_a=False, emit_ego=False)

    user_embedding = jnp.concatenate(
        [emb0[:_USER], emb1[:_USER], emb2[:_USER]], axis=1)
    item_embedding = jnp.concatenate(
        [emb0[_USER:], emb1[_USER:], emb2[_USER:]], axis=1)
    return (user_embedding, item_embedding)


# speedup vs baseline: 1.0600x; 1.0600x over previous
import functools

import jax
import jax.numpy as jnp
from jax.experimental import pallas as pl
from jax.experimental.pallas import tpu as pltpu

_USER = 4096
_N = 8192
_D = 16
_BM = 512
_NB = _N // _BM
_NBU = _USER // _BM


def _proj_body(uf_ref, u1w_ref, usf_ref, u2w_ref, itf_ref, iw_ref, out_ref):
    ue1 = jnp.dot(uf_ref[...], u1w_ref[...], preferred_element_type=jnp.float32)
    ue2 = jnp.dot(usf_ref[...], u2w_ref[...], preferred_element_type=jnp.float32)
    ie = jnp.dot(itf_ref[...], iw_ref[...], preferred_element_type=jnp.float32)
    ue = jnp.concatenate([ue1, ue2], axis=1)
    out_ref[...] = jnp.concatenate([ue, ie], axis=0).astype(jnp.bfloat16)


def _layer_body(a_ref, x_ref, w_ref, alpha_ref, emb_ref, ego_ref, abf_ref,
                *, cast_a, emit_ego):
    a = a_ref[...]
    if cast_a:
        a = a.astype(jnp.bfloat16)
        abf_ref[...] = a
    acc = jnp.dot(a, x_ref[...], preferred_element_type=jnp.float32)
    alpha = alpha_ref[0, 0]
    emb = jnp.where(acc >= 0, acc, alpha * acc)
    emb_ref[...] = emb
    if emit_ego:
        ego = jnp.dot(emb, w_ref[0], preferred_element_type=jnp.float32)
        ego_ref[...] = ego.astype(jnp.bfloat16)


def _row_spec(i):
    return (i, 0)


def _const_spec(i):
    return (0, 0)


def _w_spec(i):
    return (jnp.where(i < _NBU, 0, 1), 0, 0)


def _layer_call(a, x, w_stack, alpha, *, cast_a, emit_ego):
    in_specs = [
        pl.BlockSpec((_BM, _N), _row_spec),
        pl.BlockSpec((_N, _D), _const_spec),
    ]
    operands = [a, x]
    if emit_ego:
        in_specs.append(pl.BlockSpec((1, _D, _D), _w_spec))
        operands.append(w_stack)
    else:
        in_specs.append(pl.BlockSpec((1, _D, _D), lambda i: (0, 0, 0)))
        operands.append(jnp.zeros((1, _D, _D), jnp.float32))
    in_specs.append(pl.BlockSpec((1, 1), _const_spec))
    operands.append(alpha)

    out_shape = [jax.ShapeDtypeStruct((_N, _D), jnp.float32)]
    out_specs = [pl.BlockSpec((_BM, _D), _row_spec)]
    if emit_ego:
        out_shape.append(jax.ShapeDtypeStruct((_N, _D), jnp.bfloat16))
        out_specs.append(pl.BlockSpec((_BM, _D), _row_spec))
    if cast_a:
        out_shape.append(jax.ShapeDtypeStruct((_N, _N), jnp.bfloat16))
        out_specs.append(pl.BlockSpec((_BM, _N), _row_spec))

    def body(a_ref, x_ref, w_ref, alpha_ref, *outs):
        emb_ref = outs[0]
        ego_ref = outs[1] if emit_ego else None
        abf_ref = outs[-1] if cast_a else None
        _layer_body(a_ref, x_ref, w_ref, alpha_ref, emb_ref, ego_ref, abf_ref,
                    cast_a=cast_a, emit_ego=emit_ego)

    return pl.pallas_call(
        body,
        grid=(_NB,),
        in_specs=in_specs,
        out_specs=out_specs,
        out_shape=out_shape,
        compiler_params=pltpu.CompilerParams(
            dimension_semantics=("parallel",)),
    )(*operands)


def kernel(user_social_feat, user_feat, item_feat, raitng_adj,
           user1_w, user2_w, item_w, user_w1, item_w1, user_w2, item_w2,
           prelu_a):
    ego0 = pl.pallas_call(
        _proj_body,
        out_shape=jax.ShapeDtypeStruct((_N, _D), jnp.bfloat16),
    )(user_feat, user1_w, user_social_feat, user2_w, item_feat, item_w)

    alpha = jnp.reshape(prelu_a, (1, 1))
    w1 = jnp.stack([user_w1, item_w1])
    w2 = jnp.stack([user_w2, item_w2])

    emb0, ego1, a_bf = _layer_call(raitng_adj, ego0, w1, alpha,
                                   cast_a=True, emit_ego=True)
    emb1, ego2 = _layer_call(a_bf, ego1, w2, alpha,
                             cast_a=False, emit_ego=True)
    (emb2,) = _layer_call(a_bf, ego2, None, alpha,
                          cast_a=False, emit_ego=False)

    user_embedding = jnp.concatenate(
        [emb0[:_USER], emb1[:_USER], emb2[:_USER]], axis=1)
    item_embedding = jnp.concatenate(
        [emb0[_USER:], emb1[_USER:], emb2[_USER:]], axis=1)
    return (user_embedding, item_embedding)
